# baseline (speedup 1.0000x reference)
import functools

import jax
import jax.numpy as jnp
import numpy as np
from jax.experimental import pallas as pl
from jax.experimental.pallas import tpu as pltpu
from jax.experimental.pallas import tpu_sc as plsc

B = 16384
NUM_DENSE = 13
N_CAT = 26
VOCAB = 100000
EMB_DIM = 64
NF = N_CAT + 1
NFP = 32
BR = 512
NPAIR = N_CAT // 2
RPB = 5000
NG = 2 * B
NW = 32
PER_W = NG // NW
CHUNK = 512
N_CHUNK = PER_W // CHUNK


def _repack_pair(ta, tb):
    def body(a_ref, b_ref, o_ref):
        o_ref[...] = jnp.concatenate([a_ref[...], b_ref[...]], axis=1)

    return pl.pallas_call(
        body,
        grid=(VOCAB // RPB,),
        in_specs=[pl.BlockSpec((RPB, EMB_DIM), lambda i: (i, 0)),
                  pl.BlockSpec((RPB, EMB_DIM), lambda i: (i, 0))],
        out_specs=pl.BlockSpec((RPB, 2 * EMB_DIM), lambda i: (i, 0)),
        out_shape=jax.ShapeDtypeStruct((VOCAB, 2 * EMB_DIM), jnp.float32),
        compiler_params=pltpu.CompilerParams(
            dimension_semantics=("arbitrary",)),
    )(ta, tb)


def _sc_gather(table2, idx):
    mesh = plsc.VectorSubcoreMesh(core_axis_name="c", subcore_axis_name="s")

    @functools.partial(
        pl.kernel,
        mesh=mesh,
        out_type=jax.ShapeDtypeStruct((NG, 2 * EMB_DIM), jnp.float32),
        scratch_types=[
            pltpu.VMEM((CHUNK,), jnp.int32),
            pltpu.VMEM((CHUNK, 2 * EMB_DIM), jnp.float32),
            pltpu.SemaphoreType.DMA,
        ],
    )
    def k(table_hbm, idx_hbm, out_hbm, idx_v, rows_v, sem):
        wid = jax.lax.axis_index("s") * 2 + jax.lax.axis_index("c")
        base = wid * PER_W

        def step(i, carry):
            off = base + i * CHUNK
            pltpu.sync_copy(idx_hbm.at[pl.ds(off, CHUNK)], idx_v)
            pltpu.async_copy(table_hbm.at[idx_v], rows_v, sem).wait()
            pltpu.sync_copy(rows_v, out_hbm.at[pl.ds(off, CHUNK)])
            return carry

        jax.lax.fori_loop(0, N_CHUNK, step, 0)

    return k(table2, idx)


def _dense_body(*refs):
    (num_ref, *emb_refs) = refs[:1 + NPAIR]
    (bw0, bb0, bw1, bb1, bw2, bb2, w0bm, w0z, tb0,
     tw1, tb1, tw2, tb2, tw3, tb3, tw4, tb4, out_ref) = refs[1 + NPAIR:]
    f32, bf16 = jnp.float32, jnp.bfloat16

    def mm(a, w):
        return jax.lax.dot_general(
            a, w[...], (((1,), (0,)), ((), ())), preferred_element_type=f32)

    x = num_ref[...].astype(bf16)
    h = jnp.maximum(mm(x, bw0) + bb0[...], 0.0).astype(bf16)
    h = jnp.maximum(mm(h, bw1) + bb1[...], 0.0).astype(bf16)
    bm = jnp.maximum(mm(h, bw2) + bb2[...], 0.0)

    bm16 = bm.astype(bf16)
    los, his = [], []
    for t in range(NPAIR):
        g = emb_refs[t][...].astype(bf16)
        los.append(g[0, :, None, :EMB_DIM])
        his.append(g[1, :, None, EMB_DIM:])
    pad = jnp.zeros((BR, NFP - NF, EMB_DIM), bf16)
    feats = jnp.concatenate([bm16[:, None, :]] + los + his + [pad], axis=1)

    z = jax.lax.dot_general(feats, feats, (((2,), (2,)), ((0,), (0,))),
                            preferred_element_type=f32)
    zflat = z.astype(bf16).reshape(BR, NFP * NFP)
    hz = jax.lax.dot_general(zflat, w0z[...], (((1,), (0,)), ((), ())),
                             preferred_element_type=f32)
    h = jnp.maximum(mm(bm16, w0bm) + hz + tb0[...], 0.0).astype(bf16)
    h = jnp.maximum(mm(h, tw1) + tb1[...], 0.0).astype(bf16)
    h = jnp.maximum(mm(h, tw2) + tb2[...], 0.0).astype(bf16)
    h = jnp.maximum(mm(h, tw3) + tb3[...], 0.0).astype(bf16)
    h = mm(h, tw4) + tb4[...]
    out_ref[...] = jax.nn.sigmoid(h)


def _dense(numerical_input, embs, bw0, bb0, bw1, bb1, bw2, bb2, w0bm,
           w0z, tb0, tw1, tb1, tw2, tb2, tw3, tb3, tw4, tb4):
    n_blocks = B // BR

    def full(a):
        return pl.BlockSpec(a.shape, lambda i: tuple(0 for _ in a.shape))

    weights = (bw0, bb0, bw1, bb1, bw2, bb2, w0bm, w0z, tb0,
               tw1, tb1, tw2, tb2, tw3, tb3, tw4, tb4)
    return pl.pallas_call(
        _dense_body,
        grid=(n_blocks,),
        in_specs=[
            pl.BlockSpec((BR, NUM_DENSE), lambda i: (i, 0)),
        ] + [
            pl.BlockSpec((2, BR, 2 * EMB_DIM), lambda i: (0, i, 0))
            for _ in range(NPAIR)
        ] + [full(w) for w in weights],
        out_specs=pl.BlockSpec((BR, 1), lambda i: (i, 0)),
        out_shape=jax.ShapeDtypeStruct((B, 1), jnp.float32),
        compiler_params=pltpu.CompilerParams(
            dimension_semantics=("arbitrary",)),
    )(numerical_input, *embs, *weights)


def kernel(numerical_input, categorical_inputs, emb_tables,
           bw0, bb0, bw1, bb1, bw2, bb2,
           tw0, tb0, tw1, tb1, tw2, tb2, tw3, tb3, tw4, tb4):
    bf16 = jnp.bfloat16
    cat = categorical_inputs.astype(jnp.int32)
    embs = []
    for t in range(NPAIR):
        pt = _repack_pair(emb_tables[t], emb_tables[t + NPAIR])
        idx = jnp.concatenate([cat[:, t], cat[:, t + NPAIR]])
        embs.append(_sc_gather(pt, idx).reshape(2, B, 2 * EMB_DIM))

    li, lj = np.tril_indices(NF, -1)
    w0bm = tw0[:EMB_DIM].astype(bf16)
    w0z = jnp.zeros((NFP * NFP, tw0.shape[1]), jnp.float32)
    w0z = w0z.at[li * NFP + lj].set(tw0[EMB_DIM:]).astype(bf16)

    def row(b):
        return b.reshape(1, -1)

    return _dense(numerical_input, embs,
                  bw0.astype(bf16), row(bb0), bw1.astype(bf16), row(bb1),
                  bw2.astype(bf16), row(bb2), w0bm, w0z, row(tb0),
                  tw1.astype(bf16), row(tb1), tw2.astype(bf16), row(tb2),
                  tw3.astype(bf16), row(tb3), tw4.astype(bf16), row(tb4))


# speedup vs baseline: 1.1008x; 1.1008x over previous
import functools

import jax
import jax.numpy as jnp
import numpy as np
from jax.experimental import pallas as pl
from jax.experimental.pallas import tpu as pltpu
from jax.experimental.pallas import tpu_sc as plsc

B = 16384
NUM_DENSE = 13
N_CAT = 26
VOCAB = 100000
EMB_DIM = 64
NF = N_CAT + 1
NFP = 32
BR = 512
N_IDX = B * N_CAT
NW = 32
PER_W = N_IDX // NW
CHUNK = 832
N_CHUNK = PER_W // CHUNK


def _sc_gather(table2, idx):
    mesh = plsc.VectorSubcoreMesh(core_axis_name="c", subcore_axis_name="s")

    @functools.partial(
        pl.kernel,
        mesh=mesh,
        out_type=jax.ShapeDtypeStruct((N_IDX, 2 * EMB_DIM), jnp.float32),
        scratch_types=[
            pltpu.VMEM((CHUNK,), jnp.int32),
            pltpu.VMEM((CHUNK, 2 * EMB_DIM), jnp.float32),
            pltpu.SemaphoreType.DMA,
        ],
    )
    def k(table_hbm, idx_hbm, out_hbm, idx_v, rows_v, sem):
        wid = jax.lax.axis_index("s") * 2 + jax.lax.axis_index("c")
        base = wid * PER_W

        def step(i, carry):
            off = base + i * CHUNK
            pltpu.sync_copy(idx_hbm.at[pl.ds(off, CHUNK)], idx_v)
            pltpu.async_copy(table_hbm.at[idx_v], rows_v, sem).wait()
            pltpu.sync_copy(rows_v, out_hbm.at[pl.ds(off, CHUNK)])
            return carry

        jax.lax.fori_loop(0, N_CHUNK, step, 0)

    return k(table2, idx)


def _dense_body(num_ref, emb_ref, par_ref, bw0, bb0, bw1, bb1, bw2, bb2,
                w0bm, w0z, tb0, tw1, tb1, tw2, tb2, tw3, tb3, tw4, tb4,
                out_ref):
    f32, bf16 = jnp.float32, jnp.bfloat16

    def mm(a, w):
        return jax.lax.dot_general(
            a, w[...], (((1,), (0,)), ((), ())), preferred_element_type=f32)

    x = num_ref[...].astype(bf16)
    h = jnp.maximum(mm(x, bw0) + bb0[...], 0.0).astype(bf16)
    h = jnp.maximum(mm(h, bw1) + bb1[...], 0.0).astype(bf16)
    bm = jnp.maximum(mm(h, bw2) + bb2[...], 0.0)

    bm16 = bm.astype(bf16)
    embp = emb_ref[...]
    par = par_ref[...]
    lo, hi = embp[:, :, :EMB_DIM], embp[:, :, EMB_DIM:]
    sel = (lo + par * (hi - lo)).astype(bf16)
    pad = jnp.zeros((BR, NFP - NF, EMB_DIM), bf16)
    feats = jnp.concatenate([bm16[:, None, :], sel, pad], axis=1)

    z = jax.lax.dot_general(feats, feats, (((2,), (2,)), ((0,), (0,))),
                            preferred_element_type=f32)
    zflat = z.astype(bf16).reshape(BR, NFP * NFP)
    hz = jax.lax.dot_general(zflat, w0z[...], (((1,), (0,)), ((), ())),
                             preferred_element_type=f32)
    h = jnp.maximum(mm(bm16, w0bm) + hz + tb0[...], 0.0).astype(bf16)
    h = jnp.maximum(mm(h, tw1) + tb1[...], 0.0).astype(bf16)
    h = jnp.maximum(mm(h, tw2) + tb2[...], 0.0).astype(bf16)
    h = jnp.maximum(mm(h, tw3) + tb3[...], 0.0).astype(bf16)
    h = mm(h, tw4) + tb4[...]
    out_ref[...] = jax.nn.sigmoid(h)


def _dense(numerical_input, emb, par, bw0, bb0, bw1, bb1, bw2, bb2, w0bm,
           w0z, tb0, tw1, tb1, tw2, tb2, tw3, tb3, tw4, tb4):
    n_blocks = B // BR

    def full(a):
        return pl.BlockSpec(a.shape, lambda i: tuple(0 for _ in a.shape))

    weights = (bw0, bb0, bw1, bb1, bw2, bb2, w0bm, w0z, tb0,
               tw1, tb1, tw2, tb2, tw3, tb3, tw4, tb4)
    return pl.pallas_call(
        _dense_body,
        grid=(n_blocks,),
        in_specs=[
            pl.BlockSpec((BR, NUM_DENSE), lambda i: (i, 0)),
            pl.BlockSpec((BR, N_CAT, 2 * EMB_DIM), lambda i: (i, 0, 0)),
            pl.BlockSpec((BR, N_CAT, 1), lambda i: (i, 0, 0)),
        ] + [full(w) for w in weights],
        out_specs=pl.BlockSpec((BR, 1), lambda i: (i, 0)),
        out_shape=jax.ShapeDtypeStruct((B, 1), jnp.float32),
        compiler_params=pltpu.CompilerParams(
            dimension_semantics=("arbitrary",)),
    )(numerical_input, emb, par, *weights)


def kernel(numerical_input, categorical_inputs, emb_tables,
           bw0, bb0, bw1, bb1, bw2, bb2,
           tw0, tb0, tw1, tb1, tw2, tb2, tw3, tb3, tw4, tb4):
    bf16 = jnp.bfloat16
    cat = categorical_inputs.astype(jnp.int32)
    qoff = jnp.asarray(np.arange(N_CAT, dtype=np.int32) * VOCAB)
    q = (cat + qoff[None, :])
    par = (q & 1).astype(jnp.float32).reshape(B, N_CAT, 1)
    q2 = (q >> 1).reshape(N_IDX)
    table2 = emb_tables.reshape(N_CAT * VOCAB // 2, 2 * EMB_DIM)
    emb = _sc_gather(table2, q2).reshape(B, N_CAT, 2 * EMB_DIM)

    li, lj = np.tril_indices(NF, -1)
    w0bm = tw0[:EMB_DIM].astype(bf16)
    w0z = jnp.zeros((NFP * NFP, tw0.shape[1]), jnp.float32)
    w0z = w0z.at[li * NFP + lj].set(tw0[EMB_DIM:]).astype(bf16)

    def row(b):
        return b.reshape(1, -1)

    return _dense(numerical_input, emb, par,
                  bw0.astype(bf16), row(bb0), bw1.astype(bf16), row(bb1),
                  bw2.astype(bf16), row(bb2), w0bm, w0z, row(tb0),
                  tw1.astype(bf16), row(tb1), tw2.astype(bf16), row(tb2),
                  tw3.astype(bf16), row(tb3), tw4.astype(bf16), row(tb4))
